# Initial kernel scaffold; baseline (speedup 1.0000x reference)
import jax
import jax.numpy as jnp
from jax.experimental import pallas as pl


def kernel(x, table):
    raise NotImplementedError("write your pallas kernel here")


# speedup vs baseline: 1.8750x; 1.8750x over previous
import functools

import jax
import jax.numpy as jnp
from jax import lax
from jax.experimental import pallas as pl
from jax.experimental.pallas import tpu as pltpu
from jax.experimental.pallas import tpu_sc as plsc

VOCAB = 1000000
DIM = 64
BATCH = 16384
HIST = 50

NC = 2
NS = 16
NW = NC * NS
B_TOT = BATCH * HIST
PER_W = B_TOT // NW
CHUNK = 128
N_CHUNKS = PER_W // CHUNK
NB = 4
N_GROUPS = N_CHUNKS // NB


def _emb_body(idx_hbm, table_hbm, out_hbm, idx_v, rows_v, *sems):
    gsems = sems[:NB]
    wsems = sems[NB:]
    wid = lax.axis_index("s") * NC + lax.axis_index("c")
    base = wid * PER_W

    pltpu.sync_copy(idx_hbm.at[wid], idx_v)

    for c in range(NB - 1):
        pltpu.async_copy(table_hbm.at[idx_v.at[c]], rows_v.at[c], gsems[c])

    def group(gi, carry):
        g = gi * NB
        for b in range(NB):
            c = g + b
            pltpu.make_async_copy(
                table_hbm.at[idx_v.at[0]], rows_v.at[b], gsems[b]).wait()
            pltpu.async_copy(
                rows_v.at[b],
                out_hbm.at[pl.ds(base + c * CHUNK, CHUNK)],
                wsems[b])
            bp = (b + NB - 1) % NB
            cf = c + NB - 1

            @pl.when(cf < N_CHUNKS)
            def _prefetch():
                @pl.when(c >= 1)
                def _drain_prev_write():
                    pltpu.make_async_copy(
                        rows_v.at[bp],
                        out_hbm.at[pl.ds(base, CHUNK)],
                        wsems[bp]).wait()

                pltpu.async_copy(
                    table_hbm.at[idx_v.at[cf]], rows_v.at[bp], gsems[bp])

        return carry

    lax.fori_loop(0, N_GROUPS, group, 0)

    for b in range(NB):
        pltpu.make_async_copy(
            rows_v.at[b], out_hbm.at[pl.ds(base, CHUNK)], wsems[b]).wait()


_emb = functools.partial(
    pl.kernel,
    out_type=jax.ShapeDtypeStruct((B_TOT, DIM), jnp.float32),
    mesh=plsc.VectorSubcoreMesh(core_axis_name="c", subcore_axis_name="s"),
    scratch_types=[
        pltpu.VMEM((N_CHUNKS, CHUNK), jnp.int32),
        pltpu.VMEM((NB, CHUNK, DIM), jnp.float32),
    ] + [pltpu.SemaphoreType.DMA] * (2 * NB),
    compiler_params=pltpu.CompilerParams(use_tc_tiling_on_sc=False),
)(_emb_body)


@jax.jit
def kernel(x, table):
    idx = x.astype(jnp.int32).reshape(NW, N_CHUNKS, CHUNK)
    out = _emb(idx, table)
    return out.reshape(BATCH, HIST, DIM)
